# initial kernel scaffold (unmeasured)
import jax
import jax.numpy as jnp
from jax import lax
from jax.experimental import pallas as pl
from jax.experimental.pallas import tpu as pltpu


def kernel(
    x,
):
    def body(*refs):
        pass

    out_shape = jax.ShapeDtypeStruct(..., jnp.float32)
    return pl.pallas_call(body, out_shape=out_shape)(...)


# baseline (device time: 7307778 ns/iter reference)
import functools

import jax
import jax.numpy as jnp
from jax import lax
from jax.experimental import pallas as pl
from jax.experimental.pallas import tpu as pltpu

N_DEV = 8
M_PER = 8192
N_COLS = 1024
C_PER = N_COLS // N_DEV
L = N_DEV * M_PER
LOG_L = 16
SLAB = 8192
N_SLABS = L // SLAB


def kernel(x):
    x_bf = x.astype(jnp.bfloat16)

    def body(x_ref, out_ref, a_ref,
             send_sems1, recv_sems1, send_sems2, recv_sems2, local_sem):
        me = lax.axis_index("i")

        barrier_sem = pltpu.get_barrier_semaphore()
        for d in range(1, N_DEV):
            peer = lax.rem(me + d, N_DEV)
            pl.semaphore_signal(barrier_sem, inc=1, device_id=(peer,),
                                device_id_type=pl.DeviceIdType.MESH)
        pl.semaphore_wait(barrier_sem, N_DEV - 1)

        lcp = pltpu.make_async_copy(
            x_ref.at[:, pl.ds(me * C_PER, C_PER)],
            a_ref.at[pl.ds(me * M_PER, M_PER), :],
            local_sem,
        )
        lcp.start()
        sends = []
        for d in range(1, N_DEV):
            r = lax.rem(me + d, N_DEV)
            rd = pltpu.make_async_remote_copy(
                src_ref=x_ref.at[:, pl.ds(r * C_PER, C_PER)],
                dst_ref=a_ref.at[pl.ds(me * M_PER, M_PER), :],
                send_sem=send_sems1.at[d],
                recv_sem=recv_sems1.at[d],
                device_id=(r,),
                device_id_type=pl.DeviceIdType.MESH,
            )
            rd.start()
            sends.append(rd)
        for rd in sends:
            rd.wait_send()
        lcp.wait()
        for d in range(1, N_DEV):
            src = lax.rem(me - d + N_DEV, N_DEV)
            dummy = pltpu.make_async_remote_copy(
                src_ref=x_ref.at[:, pl.ds(src * C_PER, C_PER)],
                dst_ref=a_ref.at[pl.ds(src * M_PER, M_PER), :],
                send_sem=send_sems1.at[d],
                recv_sem=recv_sems1.at[d],
                device_id=(src,),
                device_id_type=pl.DeviceIdType.MESH,
            )
            dummy.wait_recv()

        def substep(j, k):
            @pl.when(j >= SLAB)
            def _():
                def big_body(g, carry):
                    p0 = g * SLAB
                    run = p0 // j
                    off = lax.rem(p0, j)
                    lo = pl.multiple_of(run * 2 * j + off, SLAB)
                    hi = pl.multiple_of(lo + j, SLAB)
                    a = a_ref[pl.ds(lo, SLAB), :]
                    b = a_ref[pl.ds(hi, SLAB), :]
                    asc = (lo & k) == 0
                    mn = jnp.minimum(a, b)
                    mx = jnp.maximum(a, b)
                    a_ref[pl.ds(lo, SLAB), :] = jnp.where(asc, mn, mx)
                    a_ref[pl.ds(hi, SLAB), :] = jnp.where(asc, mx, mn)
                    return carry
                lax.fori_loop(0, (L // 2) // SLAB, big_body, 0)

            @pl.when(j < SLAB)
            def _():
                def slab_body(s, carry):
                    base = pl.multiple_of(s * SLAB, SLAB)
                    xx = a_ref[pl.ds(base, SLAB), :]
                    gi = base + lax.broadcasted_iota(jnp.int32, (SLAB, 1), 0)
                    low = (gi & j) == 0
                    asc = (gi & k) == 0
                    sd = pltpu.roll(xx, SLAB - j, 0)
                    su = pltpu.roll(xx, j, 0)
                    pv = jnp.where(low, sd, su)
                    mn = jnp.minimum(xx, pv)
                    mx = jnp.maximum(xx, pv)
                    tm = low == asc
                    a_ref[pl.ds(base, SLAB), :] = jnp.where(tm, mn, mx)
                    return carry
                lax.fori_loop(0, N_SLABS, slab_body, 0)

        def stage_body(m, carry):
            k = jnp.left_shift(jnp.int32(1), m)
            def sub_body(t, c2):
                j = jnp.right_shift(k, t + 1)
                substep(j, k)
                return c2
            lax.fori_loop(0, m, sub_body, 0)
            return carry
        lax.fori_loop(1, LOG_L + 1, stage_body, 0)

        lcp2 = pltpu.make_async_copy(
            a_ref.at[pl.ds(me * M_PER, M_PER), :],
            out_ref.at[:, pl.ds(me * C_PER, C_PER)],
            local_sem,
        )
        lcp2.start()
        sends2 = []
        for d in range(1, N_DEV):
            r = lax.rem(me + d, N_DEV)
            rd = pltpu.make_async_remote_copy(
                src_ref=a_ref.at[pl.ds(r * M_PER, M_PER), :],
                dst_ref=out_ref.at[:, pl.ds(me * C_PER, C_PER)],
                send_sem=send_sems2.at[d],
                recv_sem=recv_sems2.at[d],
                device_id=(r,),
                device_id_type=pl.DeviceIdType.MESH,
            )
            rd.start()
            sends2.append(rd)
        for rd in sends2:
            rd.wait_send()
        lcp2.wait()
        for d in range(1, N_DEV):
            src = lax.rem(me - d + N_DEV, N_DEV)
            dummy = pltpu.make_async_remote_copy(
                src_ref=a_ref.at[pl.ds(me * M_PER, M_PER), :],
                dst_ref=out_ref.at[:, pl.ds(src * C_PER, C_PER)],
                send_sem=send_sems2.at[d],
                recv_sem=recv_sems2.at[d],
                device_id=(src,),
                device_id_type=pl.DeviceIdType.MESH,
            )
            dummy.wait_recv()

        @functools.partial(pl.run_scoped, sem2=pltpu.SemaphoreType.REGULAR)
        def _(sem2):
            for d in range(1, N_DEV):
                peer = lax.rem(me + d, N_DEV)
                pl.semaphore_signal(sem2, inc=1, device_id=(peer,),
                                    device_id_type=pl.DeviceIdType.MESH)
            pl.semaphore_wait(sem2, N_DEV - 1)

    return pl.pallas_call(
        body,
        out_shape=jax.ShapeDtypeStruct((M_PER, N_COLS), jnp.bfloat16),
        in_specs=[pl.BlockSpec(memory_space=pltpu.VMEM)],
        out_specs=pl.BlockSpec(memory_space=pl.ANY),
        scratch_shapes=[
            pltpu.VMEM((L, C_PER), jnp.bfloat16),
            pltpu.SemaphoreType.DMA((N_DEV,)),
            pltpu.SemaphoreType.DMA((N_DEV,)),
            pltpu.SemaphoreType.DMA((N_DEV,)),
            pltpu.SemaphoreType.DMA((N_DEV,)),
            pltpu.SemaphoreType.DMA,
        ],
        compiler_params=pltpu.CompilerParams(
            collective_id=0,
            vmem_limit_bytes=46 * 1024 * 1024,
        ),
    )(x_bf)


# device time: 1312443 ns/iter; 5.5681x vs baseline; 5.5681x over previous
import functools

import jax
import jax.numpy as jnp
from jax import lax
from jax.experimental import pallas as pl
from jax.experimental.pallas import tpu as pltpu

N_DEV = 8
M_PER = 8192
N_COLS = 1024
C_PER = N_COLS // N_DEV
L = N_DEV * M_PER
LOG_L = 16
SLAB = 8192
N_SLABS = L // SLAB


def kernel(x):
    x_bf = x.astype(jnp.bfloat16)

    def body(x_ref, out_ref, a_ref,
             send_sems1, recv_sems1, send_sems2, recv_sems2, local_sem):
        me = lax.axis_index("i")

        def _ce_reshape(xx, jj, k, base):
            nb = SLAB // (2 * jj)
            xx4 = xx.reshape(nb, 2, jj, C_PER)
            a = xx4[:, 0]
            b = xx4[:, 1]
            blk = base + lax.broadcasted_iota(
                jnp.int32, (nb, 1, 1), 0) * (2 * jj)
            asc = (blk & k) == 0
            mn = jnp.minimum(a, b)
            mx = jnp.maximum(a, b)
            na = jnp.where(asc, mn, mx)
            nbv = jnp.where(asc, mx, mn)
            return jnp.stack([na, nbv], axis=1).reshape(SLAB, C_PER)

        def _ce_word(xx, jj, k, base):
            u = pltpu.bitcast(xx, jnp.int32)
            gi = base + lax.broadcasted_iota(jnp.int32, (SLAB, 1), 0)
            low = (gi & jj) == 0
            if jj == 1:
                pv = pltpu.bitcast(
                    jnp.bitwise_or(jnp.left_shift(u, 16),
                                   lax.shift_right_logical(u, 16)),
                    jnp.bfloat16)
            else:
                w = jj // 2
                sd = pltpu.bitcast(pltpu.roll(u, SLAB // 2 - w, 0),
                                   jnp.bfloat16)
                su = pltpu.bitcast(pltpu.roll(u, w, 0),
                                   jnp.bfloat16)
                pv = jnp.where(low, sd, su)
            asc = (gi & k) == 0
            mn = jnp.minimum(xx, pv)
            mx = jnp.maximum(xx, pv)
            tm = low == asc
            return jnp.where(tm, mn, mx)

        def tail_chain(xx, k, base, gated):
            for jj in (16, 8):
                nv = _ce_reshape(xx, jj, k, base)
                xx = jnp.where((2 * jj) <= k, nv, xx) if gated else nv
            for jj in (4, 2, 1):
                nv = _ce_word(xx, jj, k, base)
                xx = jnp.where((2 * jj) <= k, nv, xx) if gated else nv
            return xx

        def mid_substep(j, k, base):
            xx = a_ref[pl.ds(base, SLAB), :]
            for jj in (32, 64, 128, 256, 512, 1024, 2048, 4096):
                @pl.when(j == jj)
                def _(jj=jj):
                    a_ref[pl.ds(base, SLAB), :] = _ce_reshape(xx, jj, k, base)

        def local_sort_slab(base):
            def stage_b(m, carry):
                k = jnp.left_shift(jnp.int32(1), m)
                def sub_b(t, c2):
                    j = jnp.right_shift(k, t + 1)
                    mid_substep(j, k, base)
                    return c2
                lax.fori_loop(0, jnp.maximum(m - 5, 0), sub_b, 0)
                @pl.when(k >= 32)
                def _():
                    xx = a_ref[pl.ds(base, SLAB), :]
                    a_ref[pl.ds(base, SLAB), :] = tail_chain(xx, k, base, False)
                @pl.when(k < 32)
                def _():
                    xx = a_ref[pl.ds(base, SLAB), :]
                    a_ref[pl.ds(base, SLAB), :] = tail_chain(xx, k, base, True)
                return carry
            lax.fori_loop(1, 14, stage_b, 0)

        barrier_sem = pltpu.get_barrier_semaphore()
        for d in range(1, N_DEV):
            peer = lax.rem(me + d, N_DEV)
            pl.semaphore_signal(barrier_sem, inc=1, device_id=(peer,),
                                device_id_type=pl.DeviceIdType.MESH)
        pl.semaphore_wait(barrier_sem, N_DEV - 1)

        lcp = pltpu.make_async_copy(
            x_ref.at[:, pl.ds(me * C_PER, C_PER)],
            a_ref.at[pl.ds(me * M_PER, M_PER), :],
            local_sem,
        )
        lcp.start()
        for d in range(1, N_DEV):
            r = lax.rem(me + d, N_DEV)
            pltpu.make_async_remote_copy(
                src_ref=x_ref.at[:, pl.ds(r * C_PER, C_PER)],
                dst_ref=a_ref.at[pl.ds(me * M_PER, M_PER), :],
                send_sem=send_sems1.at[d],
                recv_sem=recv_sems1.at[d],
                device_id=(r,),
                device_id_type=pl.DeviceIdType.MESH,
            ).start()
        lcp.wait()

        def consume(d, carry):
            src = lax.rem(me - d + N_DEV, N_DEV)
            base = pl.multiple_of(src * SLAB, SLAB)
            @pl.when(d > 0)
            def _():
                pltpu.make_async_remote_copy(
                    src_ref=x_ref.at[:, pl.ds(src * C_PER, C_PER)],
                    dst_ref=a_ref.at[pl.ds(src * M_PER, M_PER), :],
                    send_sem=send_sems1.at[d],
                    recv_sem=recv_sems1.at[d],
                    device_id=(src,),
                    device_id_type=pl.DeviceIdType.MESH,
                ).wait_recv()
            local_sort_slab(base)
            return carry
        lax.fori_loop(0, N_DEV, consume, 0)

        for d in range(1, N_DEV):
            r = lax.rem(me + d, N_DEV)
            pltpu.make_async_remote_copy(
                src_ref=x_ref.at[:, pl.ds(r * C_PER, C_PER)],
                dst_ref=a_ref.at[pl.ds(me * M_PER, M_PER), :],
                send_sem=send_sems1.at[d],
                recv_sem=recv_sems1.at[d],
                device_id=(r,),
                device_id_type=pl.DeviceIdType.MESH,
            ).wait_send()

        def gstage(m, carry):
            k = jnp.left_shift(jnp.int32(1), m)

            def sub_b(t, c2):
                j = jnp.right_shift(k, t + 1)

                @pl.when(j >= SLAB)
                def _():
                    def big_body(g, c3):
                        p0 = g * SLAB
                        run = p0 // j
                        off = lax.rem(p0, j)
                        lo = pl.multiple_of(run * 2 * j + off, SLAB)
                        hi = pl.multiple_of(lo + j, SLAB)
                        a = a_ref[pl.ds(lo, SLAB), :]
                        b = a_ref[pl.ds(hi, SLAB), :]
                        asc = (lo & k) == 0
                        mn = jnp.minimum(a, b)
                        mx = jnp.maximum(a, b)
                        a_ref[pl.ds(lo, SLAB), :] = jnp.where(asc, mn, mx)
                        a_ref[pl.ds(hi, SLAB), :] = jnp.where(asc, mx, mn)
                        return c3
                    lax.fori_loop(0, (L // 2) // SLAB, big_body, 0)

                @pl.when(j < SLAB)
                def _():
                    def slab_b(s, c3):
                        mid_substep(j, k, pl.multiple_of(s * SLAB, SLAB))
                        return c3
                    lax.fori_loop(0, N_SLABS, slab_b, 0)
                return c2
            lax.fori_loop(0, m - 5, sub_b, 0)

            def tail_slab(s, c2):
                base = pl.multiple_of(s * SLAB, SLAB)
                xx = a_ref[pl.ds(base, SLAB), :]
                a_ref[pl.ds(base, SLAB), :] = tail_chain(xx, k, base, False)

                @pl.when(m == LOG_L)
                def _():
                    @pl.when(s == me)
                    def _():
                        pltpu.make_async_copy(
                            a_ref.at[pl.ds(me * M_PER, M_PER), :],
                            out_ref.at[:, pl.ds(me * C_PER, C_PER)],
                            local_sem,
                        ).start()
                    @pl.when(s != me)
                    def _():
                        pltpu.make_async_remote_copy(
                            src_ref=a_ref.at[pl.ds(s * M_PER, M_PER), :],
                            dst_ref=out_ref.at[:, pl.ds(me * C_PER, C_PER)],
                            send_sem=send_sems2.at[s],
                            recv_sem=recv_sems2.at[me],
                            device_id=(s,),
                            device_id_type=pl.DeviceIdType.MESH,
                        ).start()
                return c2
            lax.fori_loop(0, N_SLABS, tail_slab, 0)
            return carry
        lax.fori_loop(14, LOG_L + 1, gstage, 0)

        for d in range(1, N_DEV):
            r = lax.rem(me + d, N_DEV)
            pltpu.make_async_remote_copy(
                src_ref=a_ref.at[pl.ds(r * M_PER, M_PER), :],
                dst_ref=out_ref.at[:, pl.ds(me * C_PER, C_PER)],
                send_sem=send_sems2.at[r],
                recv_sem=recv_sems2.at[me],
                device_id=(r,),
                device_id_type=pl.DeviceIdType.MESH,
            ).wait_send()
        pltpu.make_async_copy(
            a_ref.at[pl.ds(me * M_PER, M_PER), :],
            out_ref.at[:, pl.ds(me * C_PER, C_PER)],
            local_sem,
        ).wait()
        for d in range(1, N_DEV):
            src = lax.rem(me - d + N_DEV, N_DEV)
            pltpu.make_async_remote_copy(
                src_ref=a_ref.at[pl.ds(src * M_PER, M_PER), :],
                dst_ref=out_ref.at[:, pl.ds(src * C_PER, C_PER)],
                send_sem=send_sems2.at[src],
                recv_sem=recv_sems2.at[src],
                device_id=(src,),
                device_id_type=pl.DeviceIdType.MESH,
            ).wait_recv()

        @functools.partial(pl.run_scoped, sem2=pltpu.SemaphoreType.REGULAR)
        def _(sem2):
            for d in range(1, N_DEV):
                peer = lax.rem(me + d, N_DEV)
                pl.semaphore_signal(sem2, inc=1, device_id=(peer,),
                                    device_id_type=pl.DeviceIdType.MESH)
            pl.semaphore_wait(sem2, N_DEV - 1)

    return pl.pallas_call(
        body,
        out_shape=jax.ShapeDtypeStruct((M_PER, N_COLS), jnp.bfloat16),
        in_specs=[pl.BlockSpec(memory_space=pl.ANY)],
        out_specs=pl.BlockSpec(memory_space=pl.ANY),
        scratch_shapes=[
            pltpu.VMEM((L, C_PER), jnp.bfloat16),
            pltpu.SemaphoreType.DMA((N_DEV,)),
            pltpu.SemaphoreType.DMA((N_DEV,)),
            pltpu.SemaphoreType.DMA((N_DEV,)),
            pltpu.SemaphoreType.DMA((N_DEV,)),
            pltpu.SemaphoreType.DMA,
        ],
        compiler_params=pltpu.CompilerParams(
            collective_id=0,
            vmem_limit_bytes=58 * 1024 * 1024,
        ),
    )(x_bf)
